# Initial kernel scaffold; baseline (speedup 1.0000x reference)
import jax
import jax.numpy as jnp
from jax.experimental import pallas as pl


def kernel(x, edge_index, noise, W, b):
    raise NotImplementedError("write your pallas kernel here")


# speedup vs baseline: 1.9196x; 1.9196x over previous
import functools

import jax
import jax.numpy as jnp
from jax import lax
from jax.experimental import pallas as pl
from jax.experimental.pallas import tpu as pltpu
from jax.experimental.pallas import tpu_sc as plsc

N_NODES = 10000
N_EDGES = 320000
C_IN = 128
C_OUT = 128
HALF = 64

NUM_TILES = 16
NPAD = 10240
ROWS_PER_TILE = NPAD // NUM_TILES
EDGES_PER_TILE = N_EDGES // NUM_TILES
CHUNK = 80
NCHUNKS = EDGES_PER_TILE // CHUNK

_mesh = plsc.VectorSubcoreMesh(core_axis_name="c", subcore_axis_name="s")


@functools.partial(
    pl.kernel,
    mesh=_mesh,
    out_type=jax.ShapeDtypeStruct((2 * NPAD,), jnp.float32),
    scratch_types=[
        pltpu.VMEM_SHARED((NPAD,), jnp.float32),
        pltpu.VMEM((ROWS_PER_TILE,), jnp.float32),
        pltpu.VMEM((CHUNK,), jnp.int32),
        pltpu.VMEM((CHUNK,), jnp.float32),
    ],
)
def _degrees_sc(ei_hbm, deg_hbm, deg_sh, zbuf, idx_win, ones):
    c = lax.axis_index("c")
    s = lax.axis_index("s")
    zero16 = jnp.zeros((16,), jnp.float32)
    one16 = jnp.ones((16,), jnp.float32)

    def _fill_z(i, carry):
        zbuf[pl.ds(i * 16, 16)] = zero16
        return carry
    lax.fori_loop(0, ROWS_PER_TILE // 16, _fill_z, 0)
    for j in range(CHUNK // 16):
        ones[pl.ds(j * 16, 16)] = one16

    pltpu.sync_copy(zbuf, deg_sh.at[pl.ds(s * ROWS_PER_TILE, ROWS_PER_TILE)])
    plsc.subcore_barrier()

    def _chunk(k, carry):
        base = c * N_EDGES + s * EDGES_PER_TILE + k * CHUNK
        pltpu.sync_copy(ei_hbm.at[pl.ds(base, CHUNK)], idx_win)
        pltpu.sync_copy(ones, deg_sh.at[idx_win], add=True)
        return carry
    lax.fori_loop(0, NCHUNKS, _chunk, 0)

    plsc.subcore_barrier()
    pltpu.sync_copy(
        deg_sh.at[pl.ds(s * ROWS_PER_TILE, ROWS_PER_TILE)],
        deg_hbm.at[pl.ds(c * NPAD + s * ROWS_PER_TILE, ROWS_PER_TILE)])


def _h_body(x_ref, dout_ref, h_ref):
    d = dout_ref[...]
    norm = jnp.where(d > 0.0, lax.rsqrt(jnp.maximum(d, 1e-30)), 0.0)
    h_ref[...] = x_ref[...] * norm


def _h_tc(x, deg_out_col):
    bn = 2000
    return pl.pallas_call(
        _h_body,
        grid=(N_NODES // bn,),
        in_specs=[
            pl.BlockSpec((bn, C_IN), lambda i: (i, 0)),
            pl.BlockSpec((bn, 1), lambda i: (i, 0)),
        ],
        out_specs=pl.BlockSpec((bn, C_IN), lambda i: (i, 0)),
        out_shape=jax.ShapeDtypeStruct((N_NODES, C_IN), jnp.float32),
    )(x, deg_out_col)


@functools.partial(
    pl.kernel,
    mesh=_mesh,
    out_type=jax.ShapeDtypeStruct((2, NPAD, C_IN), jnp.float32),
    scratch_types=[
        pltpu.VMEM_SHARED((NPAD, C_IN), jnp.float32),
        pltpu.VMEM((128, C_IN), jnp.float32),
        pltpu.VMEM((CHUNK,), jnp.int32),
        pltpu.VMEM((CHUNK,), jnp.int32),
        pltpu.VMEM((CHUNK, C_IN), jnp.float32),
        pltpu.VMEM((CHUNK, C_IN), jnp.float32),
        pltpu.VMEM((CHUNK, C_IN), jnp.float32),
        pltpu.SemaphoreType.DMA,
        pltpu.SemaphoreType.DMA,
    ],
)
def _aggregate_sc(ei_hbm, nz_hbm, h_hbm, acc_hbm,
                  acc_sh, zbuf, edst, esrc,
                  nz_buf, h_buf, stage, sem_a, sem_b):
    c = lax.axis_index("c")
    s = lax.axis_index("s")
    zero16 = jnp.zeros((16,), jnp.float32)

    def _fill_z(i, carry):
        for u in range(C_IN // 16):
            zbuf[i, pl.ds(u * 16, 16)] = zero16
        return carry
    lax.fori_loop(0, 128, _fill_z, 0)
    for r in range(ROWS_PER_TILE // 128):
        pltpu.sync_copy(
            zbuf, acc_sh.at[pl.ds(s * ROWS_PER_TILE + r * 128, 128), :])
    plsc.subcore_barrier()

    tile_base = s * EDGES_PER_TILE

    def _chunk(k, carry):
        base = tile_base + k * CHUNK
        pltpu.sync_copy(ei_hbm.at[pl.ds(base, CHUNK)], esrc)
        pltpu.sync_copy(ei_hbm.at[pl.ds(N_EDGES + base, CHUNK)], edst)
        cp_a = pltpu.async_copy(nz_hbm.at[pl.ds(base, CHUNK), :], nz_buf,
                                sem_a)
        cp_b = pltpu.async_copy(h_hbm.at[esrc], h_buf, sem_b)
        cp_a.wait()
        cp_b.wait()

        def _edge(i, carry2):
            for u in range(HALF // 16):
                off = c * HALF + u * 16
                v = nz_buf[i, pl.ds(off, 16)]
                w = jnp.maximum(v + 1.0, 0.0)
                stage[i, pl.ds(u * 16, 16)] = w
                hv = h_buf[i, pl.ds(off, 16)]
                stage[i, pl.ds(HALF + u * 16, 16)] = w * hv
            return carry2
        lax.fori_loop(0, CHUNK, _edge, 0)

        pltpu.sync_copy(stage, acc_sh.at[edst], add=True)
        return carry
    lax.fori_loop(0, NCHUNKS, _chunk, 0)

    plsc.subcore_barrier()
    pltpu.sync_copy(acc_sh.at[pl.ds(s * ROWS_PER_TILE, ROWS_PER_TILE), :],
                    acc_hbm.at[c, pl.ds(s * ROWS_PER_TILE, ROWS_PER_TILE), :])


def _out_body(acc_ref, din_ref, w_ref, b_ref, out_ref):
    s1 = jnp.concatenate([acc_ref[0, :, :HALF], acc_ref[1, :, :HALF]], axis=1)
    s2 = jnp.concatenate([acc_ref[0, :, HALF:], acc_ref[1, :, HALF:]], axis=1)
    d = din_ref[...]
    scale = jnp.where(s1 > 0.0,
                      jnp.sqrt(d) / jnp.where(s1 > 0.0, s1, 1.0), 0.0)
    pre = scale * s2
    out_ref[...] = (
        jnp.dot(pre, w_ref[...], preferred_element_type=jnp.float32)
        + b_ref[...])


def _out_tc(acc, deg_in_col, W, b2):
    bn = 2000
    return pl.pallas_call(
        _out_body,
        grid=(N_NODES // bn,),
        in_specs=[
            pl.BlockSpec((2, bn, C_IN), lambda i: (0, i, 0)),
            pl.BlockSpec((bn, 1), lambda i: (i, 0)),
            pl.BlockSpec((C_IN, C_OUT), lambda i: (0, 0)),
            pl.BlockSpec((1, C_OUT), lambda i: (0, 0)),
        ],
        out_specs=pl.BlockSpec((bn, C_OUT), lambda i: (i, 0)),
        out_shape=jax.ShapeDtypeStruct((N_NODES, C_OUT), jnp.float32),
    )(acc, deg_in_col, W, b2)


def kernel(x, edge_index, noise, W, b):
    ei32 = edge_index.astype(jnp.int32).reshape(2 * N_EDGES)
    degs = _degrees_sc(ei32).reshape(2, NPAD)
    deg_out_col = degs[0, :N_NODES].reshape(N_NODES, 1)
    deg_in_col = degs[1, :N_NODES].reshape(N_NODES, 1)
    h2 = _h_tc(x, deg_out_col)
    acc_p = _aggregate_sc(ei32, noise, h2)
    acc = acc_p[:, :N_NODES, :]
    return _out_tc(acc, deg_in_col, W, b.reshape(1, C_OUT))


# speedup vs baseline: 2.9593x; 1.5416x over previous
import functools

import jax
import jax.numpy as jnp
from jax import lax
from jax.experimental import pallas as pl
from jax.experimental.pallas import tpu as pltpu
from jax.experimental.pallas import tpu_sc as plsc

N_NODES = 10000
N_EDGES = 320000
C_IN = 128
C_OUT = 128
HALF = 64

NUM_TILES = 16
NPAD = 10240
ROWS_PER_TILE = NPAD // NUM_TILES
EDGES_PER_TILE = N_EDGES // NUM_TILES
CHUNK = 80
NCHUNKS = EDGES_PER_TILE // CHUNK

_mesh = plsc.VectorSubcoreMesh(core_axis_name="c", subcore_axis_name="s")


@functools.partial(
    pl.kernel,
    mesh=_mesh,
    out_type=jax.ShapeDtypeStruct((2 * NPAD,), jnp.float32),
    scratch_types=[
        pltpu.VMEM_SHARED((NPAD,), jnp.float32),
        pltpu.VMEM((ROWS_PER_TILE,), jnp.float32),
        pltpu.VMEM((EDGES_PER_TILE,), jnp.int32),
        pltpu.VMEM((2, CHUNK), jnp.int32),
        pltpu.VMEM((CHUNK,), jnp.float32),
        pltpu.SemaphoreType.DMA,
        pltpu.SemaphoreType.DMA,
    ],
)
def _degrees_sc(ei_hbm, deg_hbm, deg_sh, zbuf, slab, win, ones,
                sem0, sem1):
    c = lax.axis_index("c")
    s = lax.axis_index("s")
    sems = (sem0, sem1)
    zero16 = jnp.zeros((16,), jnp.float32)
    one16 = jnp.ones((16,), jnp.float32)

    def _fill_z(i, carry):
        zbuf[pl.ds(i * 16, 16)] = zero16
        return carry
    lax.fori_loop(0, ROWS_PER_TILE // 16, _fill_z, 0)
    for j in range(CHUNK // 16):
        ones[pl.ds(j * 16, 16)] = one16

    pltpu.sync_copy(zbuf, deg_sh.at[pl.ds(s * ROWS_PER_TILE, ROWS_PER_TILE)])
    plsc.subcore_barrier()

    pltpu.sync_copy(
        ei_hbm.at[pl.ds(c * N_EDGES + s * EDGES_PER_TILE, EDGES_PER_TILE)],
        slab)

    def _fill_win(k, b):
        lo = k * CHUNK
        for j in range(CHUNK // 16):
            win[b, pl.ds(j * 16, 16)] = slab[pl.ds(lo + j * 16, 16)]

    def _issue(b):
        pltpu.async_copy(ones, deg_sh.at[win.at[b]], sems[b], add=True)

    def _wait(b):
        pltpu.make_async_copy(ones, deg_sh.at[win.at[b]], sems[b]).wait()

    for b in range(2):
        _fill_win(b, b)
        _issue(b)

    def _pair(p, carry):
        for b in range(2):
            k = 2 * p + b
            _wait(b)
            _fill_win(k, b)
            _issue(b)
        return carry
    lax.fori_loop(1, NCHUNKS // 2, _pair, 0)
    for b in range(2):
        _wait(b)

    plsc.subcore_barrier()
    pltpu.sync_copy(
        deg_sh.at[pl.ds(s * ROWS_PER_TILE, ROWS_PER_TILE)],
        deg_hbm.at[pl.ds(c * NPAD + s * ROWS_PER_TILE, ROWS_PER_TILE)])


def _h_body(x_ref, dout_ref, h_ref):
    d = dout_ref[...]
    norm = jnp.where(d > 0.0, lax.rsqrt(jnp.maximum(d, 1e-30)), 0.0)
    h_ref[...] = x_ref[...] * norm


def _h_tc(x, deg_out_col):
    bn = 2000
    return pl.pallas_call(
        _h_body,
        grid=(N_NODES // bn,),
        in_specs=[
            pl.BlockSpec((bn, C_IN), lambda i: (i, 0)),
            pl.BlockSpec((bn, 1), lambda i: (i, 0)),
        ],
        out_specs=pl.BlockSpec((bn, C_IN), lambda i: (i, 0)),
        out_shape=jax.ShapeDtypeStruct((N_NODES, C_IN), jnp.float32),
    )(x, deg_out_col)


CK = 64
EPT_SM = 19968
NC_SM = EPT_SM // CK
NC_BG = (N_EDGES - 15 * EPT_SM) // CK
NACC = 10112
ACC_ROWS_PER_TILE = NACC // NUM_TILES


@functools.partial(
    pl.kernel,
    mesh=_mesh,
    out_type=jax.ShapeDtypeStruct((2, NACC, C_IN), jnp.float32),
    scratch_types=[
        pltpu.VMEM_SHARED((NACC, C_IN), jnp.float32),
        pltpu.VMEM((2, CK), jnp.int32),
        pltpu.VMEM((2, CK), jnp.int32),
        pltpu.VMEM((2, CK), jnp.int32),
        pltpu.VMEM((2, CK, C_IN), jnp.float32),
        pltpu.VMEM((2, CK, C_IN), jnp.float32),
        pltpu.VMEM((2, CK, C_IN), jnp.float32),
        pltpu.SemaphoreType.DMA,
        pltpu.SemaphoreType.DMA,
        pltpu.SemaphoreType.DMA,
        pltpu.SemaphoreType.DMA,
        pltpu.SemaphoreType.DMA,
        pltpu.SemaphoreType.DMA,
        pltpu.SemaphoreType.DMA,
        pltpu.SemaphoreType.DMA,
    ],
)
def _aggregate_sc(ei_hbm, nz_hbm, h_hbm, acc_hbm,
                  acc_sh, swin, dwin, sidx, nz_buf, h_buf, stage,
                  sem_i0, sem_i1, sem_n0, sem_n1, sem_h0, sem_h1,
                  sem_s0, sem_s1):
    c = lax.axis_index("c")
    s = lax.axis_index("s")
    sem_i = (sem_i0, sem_i1)
    sem_n = (sem_n0, sem_n1)
    sem_h = (sem_h0, sem_h1)
    sem_s = (sem_s0, sem_s1)
    zero16 = jnp.zeros((16,), jnp.float32)
    coff = c * HALF
    tile_base = s * EPT_SM
    nc = jnp.where(s == NUM_TILES - 1, NC_BG, NC_SM)

    def _fill_z(i, carry):
        for u in range(C_IN // 16):
            stage[0, i, pl.ds(u * 16, 16)] = zero16
        return carry
    lax.fori_loop(0, CK, _fill_z, 0)
    arow = s * ACC_ROWS_PER_TILE
    for r in range(ACC_ROWS_PER_TILE // CK):
        pltpu.sync_copy(stage.at[0],
                        acc_sh.at[pl.ds(arow + r * CK, CK), :])
    rem = ACC_ROWS_PER_TILE % CK
    pltpu.sync_copy(
        stage.at[0, pl.ds(0, rem), :],
        acc_sh.at[pl.ds(arow + ACC_ROWS_PER_TILE - rem, rem), :])
    plsc.subcore_barrier()

    def _issue_idx(k, b):
        lo = tile_base + k * CK
        pltpu.async_copy(ei_hbm.at[pl.ds(lo, CK)], swin.at[b], sem_i[b])
        pltpu.async_copy(ei_hbm.at[pl.ds(N_EDGES + lo, CK)], dwin.at[b],
                         sem_i[b])

    def _wait_idx(b):
        pltpu.make_async_copy(ei_hbm.at[pl.ds(0, CK)],
                              swin.at[b], sem_i[b]).wait()
        pltpu.make_async_copy(ei_hbm.at[pl.ds(0, CK)],
                              dwin.at[b], sem_i[b]).wait()

    def _issue_gathers(k, b):
        pltpu.async_copy(nz_hbm.at[pl.ds(tile_base + k * CK, CK), :],
                         nz_buf.at[b], sem_n[b])
        pltpu.async_copy(h_hbm.at[swin.at[b]], h_buf.at[b], sem_h[b])

    def _wait_gathers(b):
        pltpu.make_async_copy(nz_hbm.at[pl.ds(0, CK), :],
                              nz_buf.at[b], sem_n[b]).wait()
        pltpu.make_async_copy(h_hbm.at[swin.at[b]], h_buf.at[b],
                              sem_h[b]).wait()

    def _compute(b):
        def _edge(i, carry2):
            for u in range(HALF // 16):
                v = nz_buf[b, i, pl.ds(coff + u * 16, 16)]
                w = jnp.maximum(v + 1.0, 0.0)
                stage[b, i, pl.ds(u * 16, 16)] = w
                hv = h_buf[b, i, pl.ds(coff + u * 16, 16)]
                stage[b, i, pl.ds(HALF + u * 16, 16)] = w * hv
            return carry2
        lax.fori_loop(0, CK, _edge, 0)

    def _issue_scatter(b):
        for j in range(CK // 16):
            sidx[b, pl.ds(j * 16, 16)] = dwin[b, pl.ds(j * 16, 16)]
        pltpu.async_copy(stage.at[b], acc_sh.at[sidx.at[b]], sem_s[b],
                         add=True)

    def _wait_scatter(b):
        pltpu.make_async_copy(stage.at[b], acc_sh.at[sidx.at[b]],
                              sem_s[b]).wait()

    def _body(k, b, first):
        _wait_idx(1 - b)
        _issue_gathers(k + 1, 1 - b)
        _wait_gathers(b)
        if not first:
            _wait_scatter(b)
        _compute(b)
        _issue_scatter(b)
        _issue_idx(k + 2, b)

    _issue_idx(0, 0)
    _issue_idx(1, 1)
    _wait_idx(0)
    _issue_gathers(0, 0)
    _body(0, 0, True)
    _body(1, 1, True)

    def _pair(p, carry):
        for b in range(2):
            _body(2 * p + b, b, False)
        return carry
    lax.fori_loop(1, nc // 2 - 1, _pair, 0)

    _wait_idx(1)
    _issue_gathers(nc - 1, 1)
    _wait_gathers(0)
    _wait_scatter(0)
    _compute(0)
    _issue_scatter(0)
    _wait_gathers(1)
    _wait_scatter(1)
    _compute(1)
    _issue_scatter(1)
    _wait_scatter(0)
    _wait_scatter(1)

    plsc.subcore_barrier()
    pltpu.sync_copy(
        acc_sh.at[pl.ds(arow, ACC_ROWS_PER_TILE), :],
        acc_hbm.at[c, pl.ds(arow, ACC_ROWS_PER_TILE), :])


def _out_body(acc_ref, din_ref, w_ref, b_ref, out_ref):
    s1 = jnp.concatenate([acc_ref[0, :, :HALF], acc_ref[1, :, :HALF]], axis=1)
    s2 = jnp.concatenate([acc_ref[0, :, HALF:], acc_ref[1, :, HALF:]], axis=1)
    d = din_ref[...]
    scale = jnp.where(s1 > 0.0,
                      jnp.sqrt(d) / jnp.where(s1 > 0.0, s1, 1.0), 0.0)
    pre = scale * s2
    out_ref[...] = (
        jnp.dot(pre, w_ref[...], preferred_element_type=jnp.float32)
        + b_ref[...])


def _out_tc(acc, deg_in_col, W, b2):
    bn = 2000
    return pl.pallas_call(
        _out_body,
        grid=(N_NODES // bn,),
        in_specs=[
            pl.BlockSpec((2, bn, C_IN), lambda i: (0, i, 0)),
            pl.BlockSpec((bn, 1), lambda i: (i, 0)),
            pl.BlockSpec((C_IN, C_OUT), lambda i: (0, 0)),
            pl.BlockSpec((1, C_OUT), lambda i: (0, 0)),
        ],
        out_specs=pl.BlockSpec((bn, C_OUT), lambda i: (i, 0)),
        out_shape=jax.ShapeDtypeStruct((N_NODES, C_OUT), jnp.float32),
    )(acc, deg_in_col, W, b2)


def kernel(x, edge_index, noise, W, b):
    ei32 = edge_index.astype(jnp.int32).reshape(2 * N_EDGES)
    degs = _degrees_sc(ei32).reshape(2, NPAD)
    deg_out_col = degs[0, :N_NODES].reshape(N_NODES, 1)
    deg_in_col = degs[1, :N_NODES].reshape(N_NODES, 1)
    h2 = _h_tc(x, deg_out_col)
    acc = _aggregate_sc(ei32, noise, h2)[:, :N_NODES, :]
    return _out_tc(acc, deg_in_col, W, b.reshape(1, C_OUT))
